# baseline (speedup 1.0000x reference)
import functools

import jax
import jax.numpy as jnp
from jax import lax
from jax.experimental import pallas as pl
from jax.experimental.pallas import tpu as pltpu
from jax.experimental.pallas import tpu_sc as plsc

_EMBED = 64
_LANES = 16
_NC = 2
_NS = 16
_NW = _NC * _NS
_CHUNK = 128
_NBUF = 8
_LEAD = 6


_IDXPAD = 8


@functools.lru_cache(maxsize=None)
def _make_idx_reorder(nb: int, nl: int):
    nrow = nb * nl // _CHUNK

    def body(x_ref, o_ref):
        o_ref[pl.ds(0, nrow), :] = (
            x_ref[...]
            .reshape(_NW, _CHUNK, nl)
            .transpose(0, 2, 1)
            .reshape(nrow, _CHUNK)
        )
        o_ref[pl.ds(nrow, (_IDXPAD - 1) * nrow), :] = jnp.zeros(
            ((_IDXPAD - 1) * nrow, _CHUNK), jnp.int32
        )

    return pl.pallas_call(
        body,
        out_shape=jax.ShapeDtypeStruct((_IDXPAD * nrow, _CHUNK), jnp.int32),
    )


@functools.lru_cache(maxsize=None)
def _make_emb_kernel(nl: int, nb: int):
    npw = nl * _CHUNK
    assert nb == _NW * _CHUNK and nl % _NBUF == 0 and nl // _NBUF >= 3
    mesh = plsc.VectorSubcoreMesh(core_axis_name="c", subcore_axis_name="s")

    @functools.partial(
        pl.kernel,
        out_type=jax.ShapeDtypeStruct((nb, nl * _EMBED), jnp.float32),
        mesh=mesh,
        scratch_types=[
            pltpu.VMEM((nl, _CHUNK), jnp.int32),
            pltpu.VMEM((_NBUF, _CHUNK, _EMBED), jnp.float32),
            pltpu.SemaphoreType.DMA((_NBUF,)),
            pltpu.SemaphoreType.DMA((_NBUF,)),
        ],
        compiler_params=pltpu.CompilerParams(use_tc_tiling_on_sc=False),
    )
    def emb(idx_hbm, table_hbm, out_hbm, idx_v, rows_v, gsem, ssem):
        wid = lax.axis_index("s") * _NC + lax.axis_index("c")
        col = wid * _CHUNK
        pltpu.sync_copy(idx_hbm.at[pl.ds(wid * nl, nl)], idx_v)

        def gather_issue(l, b):
            pltpu.async_copy(
                table_hbm.at[idx_v.at[l]],
                rows_v.at[b],
                gsem.at[b],
            )

        def gather_wait(b):
            pltpu.make_async_copy(
                table_hbm.at[pl.ds(0, _CHUNK)], rows_v.at[b], gsem.at[b]
            ).wait()

        def scatter_issue(l, b):
            pltpu.async_copy(
                rows_v.at[b],
                out_hbm.at[pl.ds(col, _CHUNK), pl.ds(l * _EMBED, _EMBED)],
                ssem.at[b],
            )

        def scatter_wait(b):
            pltpu.make_async_copy(
                rows_v.at[b],
                out_hbm.at[pl.ds(0, _CHUNK), pl.ds(0, _EMBED)],
                ssem.at[b],
            ).wait()

        def scale(b):
            @pl.loop(0, _CHUNK, unroll=8)
            def _(i):
                for j in range(_EMBED // _LANES):
                    sl = pl.ds(j * _LANES, _LANES)
                    rows_v[b, i, sl] = rows_v[b, i, sl] * 8.0

        for ls in range(_LEAD):
            gather_issue(ls, ls)

        for ls in range(_NBUF):
            gather_wait(ls)
            scale(ls)
            scatter_issue(ls, ls)
            if ls >= 2:
                scatter_wait((ls - 2) % _NBUF)
            gather_issue(ls + _LEAD, (ls + _LEAD) % _NBUF)

        @pl.loop(1, nl // _NBUF - 1)
        def _(g):
            l0 = g * _NBUF
            for ls in range(_NBUF):
                gather_wait(ls)
                scale(ls)
                scatter_issue(l0 + ls, ls)
                scatter_wait((ls + _LEAD) % _NBUF)
                gather_issue(l0 + ls + _LEAD, (ls + _LEAD) % _NBUF)

        for ls in range(_NBUF):
            gather_wait(ls)
            scale(ls)
            scatter_issue(nl - _NBUF + ls, ls)
            if ls + _LEAD < _NBUF:
                scatter_wait(ls + _LEAD)
                gather_issue(nl - _NBUF + ls + _LEAD, ls + _LEAD)

        for b in range(_NBUF):
            scatter_wait(b)

    return emb


def kernel(x, input_embedding_table):
    nb, nl = x.shape
    idxp = _make_idx_reorder(nb, nl)(x)
    out = _make_emb_kernel(nl, nb)(idxp, input_embedding_table)
    return out.reshape(nb, nl, _EMBED)


# speedup vs baseline: 1.0084x; 1.0084x over previous
import functools

import jax
import jax.numpy as jnp
from jax import lax
from jax.experimental import pallas as pl
from jax.experimental.pallas import tpu as pltpu
from jax.experimental.pallas import tpu_sc as plsc

_EMBED = 64
_LANES = 16
_NC = 2
_NS = 16
_NW = _NC * _NS
_CHUNK = 128
_NBUF = 8
_LEAD = 6


@functools.lru_cache(maxsize=None)
def _make_emb_kernel(nl: int, nb: int):
    nlt = nl // _NBUF
    assert nb == _NW * _CHUNK and nl % _NBUF == 0 and nlt >= 3
    mesh = plsc.VectorSubcoreMesh(core_axis_name="c", subcore_axis_name="s")

    @functools.partial(
        pl.kernel,
        out_type=jax.ShapeDtypeStruct((nb, nl * _EMBED), jnp.float32),
        mesh=mesh,
        scratch_types=[
            pltpu.VMEM((nlt, 1, _NBUF, _CHUNK), jnp.int32),
            pltpu.VMEM((_NBUF, _CHUNK, _EMBED), jnp.float32),
            pltpu.SemaphoreType.DMA((_NBUF,)),
            pltpu.SemaphoreType.DMA((_NBUF,)),
        ],
        compiler_params=pltpu.CompilerParams(use_tc_tiling_on_sc=False),
    )
    def emb(idx_hbm, table_hbm, out_hbm, idx_v, rows_v, gsem, ssem):
        wid = lax.axis_index("s") * _NC + lax.axis_index("c")
        col = wid * _CHUNK
        pltpu.sync_copy(idx_hbm.at[:, pl.ds(wid, 1)], idx_v)

        def gather_issue(lt, ls, b):
            pltpu.async_copy(
                table_hbm.at[idx_v.at[lt, 0, ls]], rows_v.at[b], gsem.at[b]
            )

        def gather_wait(b):
            pltpu.make_async_copy(
                table_hbm.at[pl.ds(0, _CHUNK)], rows_v.at[b], gsem.at[b]
            ).wait()

        def scatter_issue(lt, ls, b):
            l = lt * _NBUF + ls
            pltpu.async_copy(
                rows_v.at[b],
                out_hbm.at[pl.ds(col, _CHUNK), pl.ds(l * _EMBED, _EMBED)],
                ssem.at[b],
            )

        def scatter_wait(b):
            pltpu.make_async_copy(
                rows_v.at[b],
                out_hbm.at[pl.ds(0, _CHUNK), pl.ds(0, _EMBED)],
                ssem.at[b],
            ).wait()

        def scale(b):
            @pl.loop(0, _CHUNK, unroll=8)
            def _(i):
                for j in range(_EMBED // _LANES):
                    sl = pl.ds(j * _LANES, _LANES)
                    rows_v[b, i, sl] = rows_v[b, i, sl] * 8.0

        for ls in range(_LEAD):
            gather_issue(0, ls, ls)

        for ls in range(_NBUF):
            gather_wait(ls)
            scale(ls)
            scatter_issue(0, ls, ls)
            if ls >= 2:
                scatter_wait((ls - 2) % _NBUF)
            gather_issue((ls + _LEAD) // _NBUF, (ls + _LEAD) % _NBUF,
                         (ls + _LEAD) % _NBUF)

        @pl.loop(1, nlt - 1)
        def _(lt):
            for ls in range(_NBUF):
                gather_wait(ls)
                scale(ls)
                scatter_issue(lt, ls, ls)
                scatter_wait((ls + _LEAD) % _NBUF)
                gather_issue(lt + (ls + _LEAD) // _NBUF, (ls + _LEAD) % _NBUF,
                             (ls + _LEAD) % _NBUF)

        for ls in range(_NBUF):
            gather_wait(ls)
            scale(ls)
            scatter_issue(nlt - 1, ls, ls)
            if ls + _LEAD < _NBUF:
                scatter_wait(ls + _LEAD)
                gather_issue(nlt - 1, ls + _LEAD, ls + _LEAD)

        for b in range(_NBUF):
            scatter_wait(b)

    return emb


def kernel(x, input_embedding_table):
    nb, nl = x.shape
    idx4 = (
        x.T.reshape(nl // _NBUF, _NBUF, _NW, _CHUNK).transpose(0, 2, 1, 3)
    )
    out = _make_emb_kernel(nl, nb)(idx4, input_embedding_table)
    return out.reshape(nb, nl, _EMBED)
